# initial kernel scaffold (unmeasured)
import jax
import jax.numpy as jnp
from jax import lax
from jax.experimental import pallas as pl
from jax.experimental.pallas import tpu as pltpu

_CompilerParams = getattr(pltpu, "CompilerParams", None) or pltpu.TPUCompilerParams
_MESH = pltpu.DeviceIdType.MESH


def kernel(Q, K, V, bt, lens):
    B, _, H, D = Q.shape
    P_loc, BS, _, _ = K.shape
    NB = bt.shape[1]
    T = P_loc * BS
    scale = D ** -0.5

    Qt = jnp.transpose(Q[:, 0], (1, 2, 0))
    lens2 = lens.reshape(B, 1)

    def body(qt_ref, k_ref, v_ref, bt_ref, lens_ref, out_ref,
             my_acc, my_ml, pe_acc, pe_ml, send_sems, recv_sems):
        my_x = lax.axis_index("x")
        my_y = lax.axis_index("y")
        my_z = lax.axis_index("z")
        partner = (my_x, my_y, 1 - my_z)

        barrier_sem = pltpu.get_barrier_semaphore()
        pl.semaphore_signal(barrier_sem, inc=1, device_id=partner,
                            device_id_type=_MESH)
        pl.semaphore_wait(barrier_sem, 1)

        pio = lax.broadcasted_iota(jnp.int32, (P_loc, B, NB), 0) + my_z * P_loc
        jio = lax.broadcasted_iota(jnp.int32, (P_loc, B, NB), 2)
        hits = jnp.logical_and(pio == bt_ref[:][None, :, :],
                               jio < lens_ref[:][None, :, :])
        w_pi = jnp.sum(hits.astype(jnp.float32), axis=2)
        wt = jnp.broadcast_to(w_pi[:, None, :], (P_loc, BS, B)).reshape(T, B)

        for h in range(H):
            kh = k_ref[:, :, h, :].reshape(T, D).astype(jnp.bfloat16)
            qh = qt_ref[h].astype(jnp.bfloat16)
            s = lax.dot(kh, qh, preferred_element_type=jnp.float32) * scale
            m = jnp.max(s, axis=0, keepdims=True)
            p = wt * jnp.exp(s - m)
            l = jnp.sum(p, axis=0, keepdims=True)
            vh = v_ref[:, :, h, :].reshape(T, D).astype(jnp.bfloat16)
            acc = lax.dot_general(p.astype(jnp.bfloat16), vh,
                                  (((0,), (0,)), ((), ())),
                                  preferred_element_type=jnp.float32)
            my_acc[h] = acc
            my_ml[0, pl.ds(h, 1), :] = m
            my_ml[1, pl.ds(h, 1), :] = l

        rdma_a = pltpu.make_async_remote_copy(
            src_ref=my_acc, dst_ref=pe_acc,
            send_sem=send_sems.at[0], recv_sem=recv_sems.at[0],
            device_id=partner, device_id_type=_MESH)
        rdma_m = pltpu.make_async_remote_copy(
            src_ref=my_ml, dst_ref=pe_ml,
            send_sem=send_sems.at[1], recv_sem=recv_sems.at[1],
            device_id=partner, device_id_type=_MESH)
        rdma_a.start()
        rdma_m.start()
        rdma_a.wait()
        rdma_m.wait()

        m1 = my_ml[0]
        l1 = my_ml[1]
        m2 = pe_ml[0]
        l2 = pe_ml[1]
        mm = jnp.maximum(m1, m2)
        e1 = jnp.exp(m1 - mm)
        e2 = jnp.exp(m2 - mm)
        den = l1 * e1 + l2 * e2
        num = my_acc[:] * e1[:, :, None] + pe_acc[:] * e2[:, :, None]
        res = num / den[:, :, None]
        out_ref[:] = jnp.swapaxes(res, 0, 1).reshape(B, 1, H, D)

    return pl.pallas_call(
        body,
        out_shape=jax.ShapeDtypeStruct((B, 1, H, D), jnp.float32),
        in_specs=[pl.BlockSpec(memory_space=pltpu.VMEM)] * 5,
        out_specs=pl.BlockSpec(memory_space=pltpu.VMEM),
        scratch_shapes=[
            pltpu.VMEM((H, B, D), jnp.float32),
            pltpu.VMEM((2, H, B), jnp.float32),
            pltpu.VMEM((H, B, D), jnp.float32),
            pltpu.VMEM((2, H, B), jnp.float32),
            pltpu.SemaphoreType.DMA((2,)),
            pltpu.SemaphoreType.DMA((2,)),
        ],
        compiler_params=_CompilerParams(collective_id=0),
    )(Qt, K, V, bt, lens2)


# baseline (device time: 68155 ns/iter reference)
import jax
import jax.numpy as jnp
from jax import lax
from jax.experimental import pallas as pl
from jax.experimental.pallas import tpu as pltpu

_CompilerParams = getattr(pltpu, "CompilerParams", None) or pltpu.TPUCompilerParams
_MESH = pltpu.DeviceIdType.MESH


def kernel(Q, K, V, bt, lens):
    B, _, H, D = Q.shape
    P_loc, BS, _, _ = K.shape
    NB = bt.shape[1]
    T = P_loc * BS
    HD = H * D
    HB = H * B
    scale = D ** -0.5

    K2 = K.reshape(T, HD)
    V2 = V.reshape(T, HD)
    Qt = jnp.transpose(Q[:, 0], (1, 2, 0))
    lens2 = lens.reshape(B, 1)

    def body(qt_ref, k_ref, v_ref, bt_ref, lens_ref, out_ref,
             qmat, wtc, my_acc, my_ml, pe_acc, pe_ml,
             send_sems, recv_sems):
        my_x = lax.axis_index("x")
        my_y = lax.axis_index("y")
        my_z = lax.axis_index("z")
        partner = (my_x, my_y, 1 - my_z)

        barrier_sem = pltpu.get_barrier_semaphore()
        pl.semaphore_signal(barrier_sem, inc=1, device_id=partner,
                            device_id_type=_MESH)
        pl.semaphore_wait(barrier_sem, 1)

        qmat[:] = jnp.zeros((HD, HB), jnp.float32)
        for h in range(H):
            qmat[h * D:(h + 1) * D, h * B:(h + 1) * B] = qt_ref[h] * scale

        pio = lax.broadcasted_iota(jnp.int32, (P_loc, B, NB), 0) + my_z * P_loc
        jio = lax.broadcasted_iota(jnp.int32, (P_loc, B, NB), 2)
        hits = jnp.logical_and(pio == bt_ref[:][None, :, :],
                               jio < lens_ref[:][None, :, :])
        w_pi = jnp.sum(hits.astype(jnp.float32), axis=2)
        wt = jnp.broadcast_to(w_pi[:, None, :], (P_loc, BS, B)).reshape(T, B)
        for h in range(H):
            wtc[:, h * B:(h + 1) * B] = wt

        s = lax.dot(k_ref[:], qmat[:],
                    preferred_element_type=jnp.float32)
        m = jnp.max(s, axis=0, keepdims=True)
        p = wtc[:] * jnp.exp(s - m)
        l = jnp.sum(p, axis=0, keepdims=True)
        acc2 = lax.dot_general(p, v_ref[:], (((0,), (0,)), ((), ())),
                               preferred_element_type=jnp.float32)

        for h in range(H):
            my_acc[h] = acc2[h * B:(h + 1) * B, h * D:(h + 1) * D]
            my_ml[h, 0:1, :] = m[:, h * B:(h + 1) * B]
            my_ml[h, 1:2, :] = l[:, h * B:(h + 1) * B]

        rdma_a = pltpu.make_async_remote_copy(
            src_ref=my_acc, dst_ref=pe_acc,
            send_sem=send_sems.at[0], recv_sem=recv_sems.at[0],
            device_id=partner, device_id_type=_MESH)
        rdma_m = pltpu.make_async_remote_copy(
            src_ref=my_ml, dst_ref=pe_ml,
            send_sem=send_sems.at[1], recv_sem=recv_sems.at[1],
            device_id=partner, device_id_type=_MESH)
        rdma_a.start()
        rdma_m.start()
        rdma_a.wait()
        rdma_m.wait()

        m1 = my_ml[:, 0, :]
        l1 = my_ml[:, 1, :]
        m2 = pe_ml[:, 0, :]
        l2 = pe_ml[:, 1, :]
        mm = jnp.maximum(m1, m2)
        e1 = jnp.exp(m1 - mm)
        e2 = jnp.exp(m2 - mm)
        den = l1 * e1 + l2 * e2
        num = my_acc[:] * e1[:, :, None] + pe_acc[:] * e2[:, :, None]
        res = num / den[:, :, None]
        out_ref[:] = jnp.swapaxes(res, 0, 1).reshape(B, 1, H, D)

    return pl.pallas_call(
        body,
        out_shape=jax.ShapeDtypeStruct((B, 1, H, D), jnp.float32),
        in_specs=[pl.BlockSpec(memory_space=pltpu.VMEM)] * 5,
        out_specs=pl.BlockSpec(memory_space=pltpu.VMEM),
        scratch_shapes=[
            pltpu.VMEM((HD, HB), jnp.float32),
            pltpu.VMEM((T, HB), jnp.float32),
            pltpu.VMEM((H, B, D), jnp.float32),
            pltpu.VMEM((H, 2, B), jnp.float32),
            pltpu.VMEM((H, B, D), jnp.float32),
            pltpu.VMEM((H, 2, B), jnp.float32),
            pltpu.SemaphoreType.DMA((2,)),
            pltpu.SemaphoreType.DMA((2,)),
        ],
        compiler_params=_CompilerParams(
            collective_id=0, vmem_limit_bytes=100 * 1024 * 1024),
    )(Qt, K2, V2, bt, lens2)


# device time: 62075 ns/iter; 1.0979x vs baseline; 1.0979x over previous
import jax
import jax.numpy as jnp
from jax import lax
from jax.experimental import pallas as pl
from jax.experimental.pallas import tpu as pltpu

_CompilerParams = getattr(pltpu, "CompilerParams", None) or pltpu.TPUCompilerParams
_MESH = pltpu.DeviceIdType.MESH


def kernel(Q, K, V, bt, lens):
    B, _, H, D = Q.shape
    P_loc, BS, _, _ = K.shape
    NB = bt.shape[1]
    T = P_loc * BS
    HD = H * D
    HB = H * B
    scale = D ** -0.5

    K2 = K.reshape(T, HD)
    V2 = V.reshape(T, HD)
    Qt = jnp.transpose(Q[:, 0], (1, 2, 0))
    lens2 = lens.reshape(B, 1)

    def body(qt_ref, k_ref, v_ref, bt_ref, lens_ref, out_ref,
             qmat, wtc, my_acc, my_ml, pe_acc, pe_ml,
             send_sems, recv_sems):
        my_x = lax.axis_index("x")
        my_y = lax.axis_index("y")
        my_z = lax.axis_index("z")
        partner = (my_x, my_y, 1 - my_z)

        barrier_sem = pltpu.get_barrier_semaphore()
        pl.semaphore_signal(barrier_sem, inc=1, device_id=partner,
                            device_id_type=_MESH)
        pl.semaphore_wait(barrier_sem, 1)

        qmat[:] = jnp.zeros((HD, HB), jnp.bfloat16)
        for h in range(H):
            qmat[h * D:(h + 1) * D, h * B:(h + 1) * B] = (
                qt_ref[h] * scale).astype(jnp.bfloat16)

        pio = lax.broadcasted_iota(jnp.int32, (P_loc, B, NB), 0) + my_z * P_loc
        jio = lax.broadcasted_iota(jnp.int32, (P_loc, B, NB), 2)
        hits = jnp.logical_and(pio == bt_ref[:][None, :, :],
                               jio < lens_ref[:][None, :, :])
        w_pi = jnp.sum(hits.astype(jnp.float32), axis=2)
        wt = jnp.broadcast_to(w_pi[:, None, :], (P_loc, BS, B)).reshape(T, B)
        for h in range(H):
            wtc[:, h * B:(h + 1) * B] = wt

        s = lax.dot(k_ref[:].astype(jnp.bfloat16), qmat[:],
                    preferred_element_type=jnp.float32)
        m = jnp.max(s, axis=0, keepdims=True)
        p = wtc[:] * jnp.exp(s - m)
        l = jnp.sum(p, axis=0, keepdims=True)
        acc2 = lax.dot_general(p.astype(jnp.bfloat16),
                               v_ref[:].astype(jnp.bfloat16),
                               (((0,), (0,)), ((), ())),
                               preferred_element_type=jnp.float32)

        for h in range(H):
            my_acc[h] = acc2[h * B:(h + 1) * B, h * D:(h + 1) * D]
            my_ml[h, 0:1, :] = m[:, h * B:(h + 1) * B]
            my_ml[h, 1:2, :] = l[:, h * B:(h + 1) * B]

        rdma_a = pltpu.make_async_remote_copy(
            src_ref=my_acc, dst_ref=pe_acc,
            send_sem=send_sems.at[0], recv_sem=recv_sems.at[0],
            device_id=partner, device_id_type=_MESH)
        rdma_m = pltpu.make_async_remote_copy(
            src_ref=my_ml, dst_ref=pe_ml,
            send_sem=send_sems.at[1], recv_sem=recv_sems.at[1],
            device_id=partner, device_id_type=_MESH)
        rdma_a.start()
        rdma_m.start()
        rdma_a.wait()
        rdma_m.wait()

        m1 = my_ml[:, 0, :]
        l1 = my_ml[:, 1, :]
        m2 = pe_ml[:, 0, :]
        l2 = pe_ml[:, 1, :]
        mm = jnp.maximum(m1, m2)
        e1 = jnp.exp(m1 - mm)
        e2 = jnp.exp(m2 - mm)
        den = l1 * e1 + l2 * e2
        num = my_acc[:] * e1[:, :, None] + pe_acc[:] * e2[:, :, None]
        res = num / den[:, :, None]
        out_ref[:] = jnp.swapaxes(res, 0, 1).reshape(B, 1, H, D)

    return pl.pallas_call(
        body,
        out_shape=jax.ShapeDtypeStruct((B, 1, H, D), jnp.float32),
        in_specs=[pl.BlockSpec(memory_space=pltpu.VMEM)] * 5,
        out_specs=pl.BlockSpec(memory_space=pltpu.VMEM),
        scratch_shapes=[
            pltpu.VMEM((HD, HB), jnp.bfloat16),
            pltpu.VMEM((T, HB), jnp.float32),
            pltpu.VMEM((H, B, D), jnp.float32),
            pltpu.VMEM((H, 2, B), jnp.float32),
            pltpu.VMEM((H, B, D), jnp.float32),
            pltpu.VMEM((H, 2, B), jnp.float32),
            pltpu.SemaphoreType.DMA((2,)),
            pltpu.SemaphoreType.DMA((2,)),
        ],
        compiler_params=_CompilerParams(
            collective_id=0, vmem_limit_bytes=100 * 1024 * 1024),
    )(Qt, K2, V2, bt, lens2)


# device time: 57776 ns/iter; 1.1796x vs baseline; 1.0744x over previous
import jax
import jax.numpy as jnp
from jax import lax
from jax.experimental import pallas as pl
from jax.experimental.pallas import tpu as pltpu

_CompilerParams = getattr(pltpu, "CompilerParams", None) or pltpu.TPUCompilerParams
_MESH = pltpu.DeviceIdType.MESH

_CHUNKS = 4
_SHIFT = 20.0
_BIG = 30.0
_LOGW0 = -60.0


def kernel(Q, K, V, bt, lens):
    B, _, H, D = Q.shape
    P_loc, BS, _, _ = K.shape
    NB = bt.shape[1]
    HB = H * B
    PG = P_loc // _CHUNKS
    RCH = PG * BS * H
    AUG = D + H + 1 + B
    scale = D ** -0.5

    Qs = jnp.transpose(Q[:, 0] * scale, (2, 1, 0)).reshape(D, HB)
    c = jnp.arange(HB)
    hsel = (jnp.arange(H)[:, None] == (c // B)[None, :]).astype(jnp.float32)
    isel = (jnp.arange(B)[:, None] == (c % B)[None, :]).astype(jnp.float32)
    crow = jnp.full((1, HB), -(_BIG + _SHIFT), jnp.float32)
    Qa = jnp.concatenate([Qs, hsel, crow, isel], axis=0)
    lens2 = lens.reshape(B, 1)
    Kb = K.astype(jnp.bfloat16)
    Vb = V.astype(jnp.bfloat16)

    def body(qa_ref, k_ref, v_ref, bt_ref, lens_ref, out_ref,
             my_s, pe_s, send_sem, recv_sem):
        my_x = lax.axis_index("x")
        my_y = lax.axis_index("y")
        my_z = lax.axis_index("z")
        partner = (my_x, my_y, 1 - my_z)

        barrier_sem = pltpu.get_barrier_semaphore()
        pl.semaphore_signal(barrier_sem, inc=1, device_id=partner,
                            device_id_type=_MESH)
        pl.semaphore_wait(barrier_sem, 1)

        pio = lax.broadcasted_iota(jnp.int32, (P_loc, B, NB), 0) + my_z * P_loc
        jio = lax.broadcasted_iota(jnp.int32, (P_loc, B, NB), 2)
        hits = jnp.logical_and(pio == bt_ref[:][None, :, :],
                               jio < lens_ref[:][None, :, :])
        w_pi = jnp.sum(hits.astype(jnp.float32), axis=2)
        lw_pi = jnp.where(w_pi > 0, jnp.log(jnp.maximum(w_pi, 1.0)), _LOGW0)

        hfeat = (_BIG * (lax.broadcasted_iota(jnp.int32, (RCH, H), 0) % H ==
                         lax.broadcasted_iota(jnp.int32, (RCH, H), 1))
                 .astype(jnp.float32)).astype(jnp.bfloat16)
        ones_col = jnp.ones((RCH, 1), jnp.bfloat16)
        qab = qa_ref[:].astype(jnp.bfloat16)

        acc = jnp.zeros((HB, D + 1), jnp.float32)
        for ch in range(_CHUNKS):
            kc = k_ref[ch * PG:(ch + 1) * PG].reshape(RCH, D)
            vc = v_ref[ch * PG:(ch + 1) * PG].reshape(RCH, D)
            lwr = jnp.broadcast_to(
                lw_pi[ch * PG:(ch + 1) * PG][:, None, :],
                (PG, BS * H, B)).reshape(RCH, B)
            ka = jnp.concatenate(
                [kc, hfeat, ones_col, lwr.astype(jnp.bfloat16)],
                axis=1)
            s = lax.dot(ka, qab, preferred_element_type=jnp.float32)
            p = jnp.exp(s).astype(jnp.bfloat16)
            vplus = jnp.concatenate([vc, ones_col], axis=1)
            acc = acc + lax.dot_general(p, vplus, (((0,), (0,)), ((), ())),
                                        preferred_element_type=jnp.float32)
        my_s[:] = acc

        rdma = pltpu.make_async_remote_copy(
            src_ref=my_s, dst_ref=pe_s,
            send_sem=send_sem, recv_sem=recv_sem,
            device_id=partner, device_id_type=_MESH)
        rdma.start()
        rdma.wait()

        tot = my_s[:] + pe_s[:]
        res = tot[:, 0:D] / tot[:, D:D + 1]
        for h in range(H):
            out_ref[:, 0, h, :] = res[h * B:(h + 1) * B, :]

    return pl.pallas_call(
        body,
        out_shape=jax.ShapeDtypeStruct((B, 1, H, D), jnp.float32),
        in_specs=[pl.BlockSpec(memory_space=pltpu.VMEM)] * 5,
        out_specs=pl.BlockSpec(memory_space=pltpu.VMEM),
        scratch_shapes=[
            pltpu.VMEM((HB, D + 1), jnp.float32),
            pltpu.VMEM((HB, D + 1), jnp.float32),
            pltpu.SemaphoreType.DMA,
            pltpu.SemaphoreType.DMA,
        ],
        compiler_params=_CompilerParams(
            collective_id=0, vmem_limit_bytes=100 * 1024 * 1024),
    )(Qa, Kb, Vb, bt, lens2)
